# baseline (speedup 1.0000x reference)
import functools

import jax
import jax.numpy as jnp
import numpy as np
from jax import lax
from jax.experimental import pallas as pl
from jax.experimental.pallas import tpu as pltpu
from jax.experimental.pallas import tpu_sc as plsc

N = 10000
E = 160000
NC = 40

C_BN = float(1.0 / np.sqrt(1.0 + 1e-5))
C6 = 0.6 * C_BN
C4 = 0.4 * C_BN

EBLK = 3200
NBLK = 1000
KSC = 64
NWORK = 32
NPAD = 10048


def _lr(v):
    return jnp.where(v >= 0, v, 0.2 * v)


def _make_sc_edge(din, dout, cs_copy):
    w = din + dout
    wp = -(-w // 128) * 128
    nch = E // KSC
    iters = -(-nch // NWORK)
    mesh = plsc.VectorSubcoreMesh(core_axis_name="c", subcore_axis_name="s")

    @functools.partial(
        pl.kernel,
        out_type=jax.ShapeDtypeStruct((E, wp), jnp.float32),
        mesh=mesh,
        scratch_types=[
            pltpu.VMEM((KSC,), jnp.int32),
            pltpu.VMEM((KSC,), jnp.int32),
            pltpu.VMEM((KSC,), jnp.int32),
            pltpu.VMEM((KSC,), jnp.int32),
            pltpu.VMEM((KSC, wp), jnp.float32),
            pltpu.VMEM((KSC, wp), jnp.float32),
            pltpu.VMEM((KSC, wp), jnp.float32),
            pltpu.VMEM((KSC, wp), jnp.float32),
            pltpu.SemaphoreType.DMA,
            pltpu.SemaphoreType.DMA,
            pltpu.SemaphoreType.DMA,
            pltpu.SemaphoreType.DMA,
        ],
    )
    def k(t_hbm, rs_hbm, cs_hbm, out_hbm,
          ir0, ic0, ir1, ic1, br0, bc0, br1, bc1, sr0, sc0, sr1, sc1):
        wid = lax.axis_index("s") * 2 + lax.axis_index("c")
        slots = ((ir0, ic0, br0, bc0, sr0, sc0),
                 (ir1, ic1, br1, bc1, sr1, sc1))

        def issue(i, slot):
            ir, ic, br, bc, sr, sc_ = slot
            cidx = wid + NWORK * i

            @pl.when(cidx < nch)
            def _():
                base = cidx * KSC
                pltpu.sync_copy(rs_hbm.at[pl.ds(base, KSC)], ir)
                pltpu.sync_copy(cs_hbm.at[pl.ds(base, KSC)], ic)
                pltpu.async_copy(t_hbm.at[ir], br, sr)
                pltpu.async_copy(t_hbm.at[ic], bc, sc_)

        def finish(i, slot):
            ir, ic, br, bc, sr, sc_ = slot
            cidx = wid + NWORK * i

            @pl.when(cidx < nch)
            def _():
                base = cidx * KSC
                pltpu.make_async_copy(t_hbm.at[ir], br, sr).wait()
                pltpu.make_async_copy(t_hbm.at[ic], bc, sc_).wait()

                def sub(j, c):
                    for f in range(w // 16):
                        sl = pl.ds(f * 16, 16)
                        d = br[j, sl] - bc[j, sl]
                        if (f + 1) * 16 <= din:
                            d = jnp.abs(d)
                        br[j, sl] = d
                    for f in cs_copy:
                        sl = pl.ds(f * 16, 16)
                        br[j, sl] = bc[j, sl]
                    return c

                lax.fori_loop(0, KSC, sub, 0)
                pltpu.sync_copy(br, out_hbm.at[pl.ds(base, KSC)])

        issue(0, slots[0])

        def body(i2, carry):
            i = 2 * i2
            issue(i + 1, slots[1])
            finish(i, slots[0])
            issue(i + 2, slots[0])
            finish(i + 1, slots[1])
            return carry

        lax.fori_loop(0, (iters + 1) // 2, body, 0)

    return k


def _sc_edge(t, rs, cs, din, dout, cs_copy=()):
    return _make_sc_edge(din, dout, cs_copy)(t, rs, cs)


def _make_sc_rowgather(mp):
    nch = NPAD // KSC
    iters = -(-nch // NWORK)
    mesh = plsc.VectorSubcoreMesh(core_axis_name="c", subcore_axis_name="s")

    @functools.partial(
        pl.kernel,
        out_type=jax.ShapeDtypeStruct((NPAD, mp), jnp.float32),
        mesh=mesh,
        scratch_types=[
            pltpu.VMEM((KSC,), jnp.int32),
            pltpu.VMEM((KSC, mp), jnp.float32),
            pltpu.SemaphoreType.DMA,
        ],
    )
    def k(src_hbm, idx_hbm, out_hbm, iv, buf, sem):
        wid = lax.axis_index("s") * 2 + lax.axis_index("c")

        def body(i, carry):
            cidx = wid + NWORK * i

            @pl.when(cidx < nch)
            def _():
                base = cidx * KSC
                pltpu.sync_copy(idx_hbm.at[pl.ds(base, KSC)], iv)
                pltpu.async_copy(src_hbm.at[iv], buf, sem).wait()
                pltpu.sync_copy(buf, out_hbm.at[pl.ds(base, KSC)])

            return carry

        lax.fori_loop(0, iters, body, 0)

    return k


def _sc_rowgather(src, idx):
    return _make_sc_rowgather(src.shape[1])(src, idx)


def _prep1_body(din, dout, wp, hn_ref, wt_ref, wb_ref, ew_ref, st_ref,
                t_ref, a_ref):
    hn = hn_ref[...]
    t_ref[:, :din] = hn
    t_ref[:, din:din + dout] = jnp.dot(hn, wb_ref[...],
                                       preferred_element_type=jnp.float32)
    nb = hn.shape[0]
    t_ref[:, din + dout:din + dout + 16] = jnp.broadcast_to(
        ew_ref[...], (nb, 16))
    t_ref[:, din + dout + 16:din + dout + 32] = jnp.broadcast_to(
        st_ref[...], (nb, 16))
    if wp > din + dout + 32:
        t_ref[:, din + dout + 32:] = jnp.zeros(
            (nb, wp - din - dout - 32), jnp.float32)
    a_ref[...] = jnp.dot(_lr(C_BN * hn), wt_ref[...],
                         preferred_element_type=jnp.float32)


def _prep1(hn, wt, wb, ew1, st1):
    din, dout = wt.shape
    wp = -(-(din + dout + 32) // 128) * 128
    grid = N // NBLK
    return pl.pallas_call(
        functools.partial(_prep1_body, din, dout, wp),
        grid=(grid,),
        in_specs=[
            pl.BlockSpec((NBLK, din), lambda i: (i, 0)),
            pl.BlockSpec((din, dout), lambda i: (0, 0)),
            pl.BlockSpec((din, dout), lambda i: (0, 0)),
            pl.BlockSpec((NBLK, 1), lambda i: (i, 0)),
            pl.BlockSpec((NBLK, 1), lambda i: (i, 0)),
        ],
        out_specs=[
            pl.BlockSpec((NBLK, wp), lambda i: (i, 0)),
            pl.BlockSpec((NBLK, dout), lambda i: (i, 0)),
        ],
        out_shape=[
            jax.ShapeDtypeStruct((N, wp), jnp.float32),
            jax.ShapeDtypeStruct((N, dout), jnp.float32),
        ],
    )(hn, wt, wb, ew1, st1)


def _prep_agg_body(din, dout, wp, ap_ref, m_ref, cnt_ref, wt_ref, wb_ref,
                   t_ref, a_ref, agg_ref):
    hn = jnp.where(cnt_ref[...] > 0, ap_ref[...] + m_ref[...], 0.0)
    agg_ref[...] = hn
    t_ref[:, :din] = hn
    t_ref[:, din:din + dout] = jnp.dot(hn, wb_ref[...],
                                       preferred_element_type=jnp.float32)
    if wp > din + dout:
        t_ref[:, din + dout:] = jnp.zeros(
            (hn.shape[0], wp - din - dout), jnp.float32)
    a_ref[...] = jnp.dot(_lr(C_BN * hn), wt_ref[...],
                         preferred_element_type=jnp.float32)


def _prep_agg(aprev, mpad, cntf, wt, wb):
    din, dout = wt.shape
    wp = -(-(din + dout) // 128) * 128
    grid = N // NBLK
    return pl.pallas_call(
        functools.partial(_prep_agg_body, din, dout, wp),
        grid=(grid,),
        in_specs=[
            pl.BlockSpec((NBLK, din), lambda i: (i, 0)),
            pl.BlockSpec((NBLK, din), lambda i: (i, 0)),
            pl.BlockSpec((NBLK, 1), lambda i: (i, 0)),
            pl.BlockSpec((din, dout), lambda i: (0, 0)),
            pl.BlockSpec((din, dout), lambda i: (0, 0)),
        ],
        out_specs=[
            pl.BlockSpec((NBLK, wp), lambda i: (i, 0)),
            pl.BlockSpec((NBLK, dout), lambda i: (i, 0)),
            pl.BlockSpec((NBLK, din), lambda i: (i, 0)),
        ],
        out_shape=[
            jax.ShapeDtypeStruct((N, wp), jnp.float32),
            jax.ShapeDtypeStruct((N, dout), jnp.float32),
            jax.ShapeDtypeStruct((N, din), jnp.float32),
        ],
    )(aprev, mpad, cntf, wt, wb)


def _edge_body(din, dout, mp, dxdu_ref, aux_ref, wb_ref, out_ref, carry_h):
    i = pl.program_id(0)

    @pl.when(i == 0)
    def _():
        carry_h[...] = jnp.zeros_like(carry_h)

    blk = dxdu_ref[...]
    dx = jnp.abs(blk[:, :din])
    du = blk[:, din:din + dout]
    ew = aux_ref[:, 0:1]
    startf = aux_ref[:, 1:2]
    m = ew * (C6 * du + C4 * jnp.dot(
        dx, wb_ref[...], preferred_element_type=jnp.float32))

    rowi = lax.broadcasted_iota(jnp.int32, (EBLK, 1), 0).astype(jnp.float32)
    gbase = (i * EBLK).astype(jnp.float32)
    posv = rowi + gbase - startf
    pr = jnp.minimum(posv, rowi)

    s = 1
    while s < EBLK:
        hp = pltpu.roll(m, s, 0)
        m = jnp.where(pr >= s, jnp.maximum(m, hp), m)
        s *= 2

    m = jnp.where(posv > rowi, jnp.maximum(m, carry_h[...]), m)

    carry_h[...] = m[EBLK - 1:EBLK, :]
    out_ref[:, :dout] = m
    if mp > dout:
        out_ref[:, dout:] = jnp.zeros((EBLK, mp - dout), jnp.float32)


def _edge_scan(dxdu, aux, wb):
    din, dout = wb.shape
    wp = dxdu.shape[1]
    mp = max(dout, 128)
    grid = E // EBLK
    return pl.pallas_call(
        functools.partial(_edge_body, din, dout, mp),
        grid=(grid,),
        in_specs=[
            pl.BlockSpec((EBLK, wp), lambda i: (i, 0)),
            pl.BlockSpec((EBLK, 2), lambda i: (i, 0)),
            pl.BlockSpec((din, dout), lambda i: (0, 0)),
        ],
        out_specs=pl.BlockSpec((EBLK, mp), lambda i: (i, 0)),
        out_shape=jax.ShapeDtypeStruct((E, mp), jnp.float32),
        scratch_shapes=[
            pltpu.VMEM((1, dout), jnp.float32),
        ],
    )(dxdu, aux, wb)


def _joint_body(a1, a2, a3, a4p, m4, cnt, w1, w2, w3, w4, wl1, wl2, wl3,
                cl1, cl2, cl3, out_ref, mx_s, sm_s):
    i = pl.program_id(0)
    nb = pl.num_programs(0)
    agg4 = jnp.where(cnt[...] > 0, a4p[...] + m4[...], 0.0)
    hj = (jnp.dot(_lr(C_BN * a1[...]), w1[...],
                  preferred_element_type=jnp.float32)
          + jnp.dot(_lr(C_BN * a2[...]), w2[...],
                    preferred_element_type=jnp.float32)
          + jnp.dot(_lr(C_BN * a3[...]), w3[...],
                    preferred_element_type=jnp.float32)
          + jnp.dot(_lr(C_BN * agg4), w4[...],
                    preferred_element_type=jnp.float32))
    bm = jnp.max(hj, axis=0, keepdims=True)
    bs = jnp.sum(hj, axis=0, keepdims=True)

    @pl.when(i == 0)
    def _():
        mx_s[...] = bm
        sm_s[...] = bs

    @pl.when(i > 0)
    def _():
        mx_s[...] = jnp.maximum(mx_s[...], bm)
        sm_s[...] = sm_s[...] + bs

    @pl.when(i == nb - 1)
    def _():
        g = jnp.concatenate([mx_s[...], sm_s[...] * (1.0 / N)], axis=1)
        g = jnp.dot(_lr(C_BN * g), wl1[...],
                    preferred_element_type=jnp.float32) + cl1[...]
        g = jnp.dot(_lr(C_BN * g), wl2[...],
                    preferred_element_type=jnp.float32) + cl2[...]
        g = jnp.dot(_lr(C_BN * g), wl3[...],
                    preferred_element_type=jnp.float32) + cl3[...]
        out_ref[...] = g


def _joint(aggs, a4, m4pad, cntf, wjs, p):
    grid = N // NBLK
    douts = [a.shape[1] for a in aggs] + [a4.shape[1]]
    in_specs = [pl.BlockSpec((NBLK, d), lambda i: (i, 0)) for d in douts]
    in_specs += [
        pl.BlockSpec((NBLK, a4.shape[1]), lambda i: (i, 0)),
        pl.BlockSpec((NBLK, 1), lambda i: (i, 0)),
    ]
    in_specs += [pl.BlockSpec((d, 1024), lambda i: (0, 0)) for d in douts]
    in_specs += [
        pl.BlockSpec((2048, 512), lambda i: (0, 0)),
        pl.BlockSpec((512, 256), lambda i: (0, 0)),
        pl.BlockSpec((256, NC), lambda i: (0, 0)),
        pl.BlockSpec((1, 512), lambda i: (0, 0)),
        pl.BlockSpec((1, 256), lambda i: (0, 0)),
        pl.BlockSpec((1, NC), lambda i: (0, 0)),
    ]
    return pl.pallas_call(
        _joint_body,
        grid=(grid,),
        in_specs=in_specs,
        out_specs=pl.BlockSpec((1, NC), lambda i: (0, 0)),
        out_shape=jax.ShapeDtypeStruct((1, NC), jnp.float32),
        scratch_shapes=[
            pltpu.VMEM((1, 1024), jnp.float32),
            pltpu.VMEM((1, 1024), jnp.float32),
        ],
    )(*aggs, a4, m4pad, cntf, *wjs,
      p['Wl1'], p['Wl2'], p['Wl3'],
      p['cl1'].reshape(1, -1), p['cl2'].reshape(1, -1),
      p['cl3'].reshape(1, -1))


def kernel(x, params, edge_index):
    p = params
    row = edge_index[0].astype(jnp.int32)
    col = edge_index[1].astype(jnp.int32)

    deg = jax.ops.segment_sum(jnp.ones((E,), jnp.float32), row,
                              num_segments=N)
    ewn = (1.0 / jnp.maximum(deg, 1.0))[:, None]

    cs, rs = lax.sort((col, row), num_keys=1)
    cnt = jax.ops.segment_sum(jnp.ones((E,), jnp.int32), cs,
                              num_segments=N, indices_are_sorted=True)
    end = jnp.cumsum(cnt)
    startf = (end - cnt).astype(jnp.float32)[:, None]
    cntf = cnt.astype(jnp.float32)[:, None]
    idxend = jnp.concatenate(
        [jnp.clip(end - 1, 0, E - 1),
         jnp.zeros((NPAD - N,), jnp.int32)])

    dins = (256, 64, 64, 128)
    wkeys = ('W1', 'W2', 'W3', 'W4')

    w = p['W1']
    wt, wb = w[:256], w[256:]
    t, a = _prep1(x, wt, wb, ewn, startf)
    dxdu = _sc_edge(t, rs, cs, 256, 64, cs_copy=(21,))
    aux = jnp.concatenate([dxdu[:, 320:321], dxdu[:, 336:337]], axis=1)
    mscan = _edge_scan(dxdu, aux, wb)
    mpad = _sc_rowgather(mscan, idxend)
    aggs = []
    aprev = a

    for li in (1, 2, 3):
        w = p[wkeys[li]]
        din = dins[li]
        wt, wb = w[:din], w[din:]
        t, a, agg = _prep_agg(aprev, mpad[:N, :din], cntf, wt, wb)
        aggs.append(agg)
        dxdu = _sc_edge(t, rs, cs, din, wb.shape[1])
        mscan = _edge_scan(dxdu, aux, wb)
        mpad = _sc_rowgather(mscan, idxend)
        aprev = a

    wj = p['Wj']
    wjs = [wj[0:64], wj[64:128], wj[128:256], wj[256:512]]
    return _joint(aggs, aprev, mpad, cntf, wjs, p)


# speedup vs baseline: 1.0167x; 1.0167x over previous
import functools

import jax
import jax.numpy as jnp
import numpy as np
from jax import lax
from jax.experimental import pallas as pl
from jax.experimental.pallas import tpu as pltpu
from jax.experimental.pallas import tpu_sc as plsc

N = 10000
E = 160000
NC = 40

C_BN = float(1.0 / np.sqrt(1.0 + 1e-5))
C6 = 0.6 * C_BN
C4 = 0.4 * C_BN

EBLK = 1280
NBLK = 1000
KSC = 64
NWORK = 32
NPAD = 10048


def _lr(v):
    return jnp.where(v >= 0, v, 0.2 * v)


def _make_sc_edge(din, dout, cs_copy):
    w = din + dout
    wp = -(-w // 128) * 128
    nch = E // KSC
    iters = -(-nch // NWORK)
    mesh = plsc.VectorSubcoreMesh(core_axis_name="c", subcore_axis_name="s")

    @functools.partial(
        pl.kernel,
        out_type=jax.ShapeDtypeStruct((E, wp), jnp.float32),
        mesh=mesh,
        scratch_types=[
            pltpu.VMEM((KSC,), jnp.int32),
            pltpu.VMEM((KSC,), jnp.int32),
            pltpu.VMEM((KSC,), jnp.int32),
            pltpu.VMEM((KSC,), jnp.int32),
            pltpu.VMEM((KSC, wp), jnp.float32),
            pltpu.VMEM((KSC, wp), jnp.float32),
            pltpu.VMEM((KSC, wp), jnp.float32),
            pltpu.VMEM((KSC, wp), jnp.float32),
            pltpu.SemaphoreType.DMA,
            pltpu.SemaphoreType.DMA,
            pltpu.SemaphoreType.DMA,
            pltpu.SemaphoreType.DMA,
        ],
    )
    def k(t_hbm, rs_hbm, cs_hbm, out_hbm,
          ir0, ic0, ir1, ic1, br0, bc0, br1, bc1, sr0, sc0, sr1, sc1):
        wid = lax.axis_index("s") * 2 + lax.axis_index("c")
        slots = ((ir0, ic0, br0, bc0, sr0, sc0),
                 (ir1, ic1, br1, bc1, sr1, sc1))

        def issue(i, slot):
            ir, ic, br, bc, sr, sc_ = slot
            cidx = wid + NWORK * i

            @pl.when(cidx < nch)
            def _():
                base = cidx * KSC
                pltpu.sync_copy(rs_hbm.at[pl.ds(base, KSC)], ir)
                pltpu.sync_copy(cs_hbm.at[pl.ds(base, KSC)], ic)
                pltpu.async_copy(t_hbm.at[ir], br, sr)
                pltpu.async_copy(t_hbm.at[ic], bc, sc_)

        def finish(i, slot):
            ir, ic, br, bc, sr, sc_ = slot
            cidx = wid + NWORK * i

            @pl.when(cidx < nch)
            def _():
                base = cidx * KSC
                pltpu.make_async_copy(t_hbm.at[ir], br, sr).wait()
                pltpu.make_async_copy(t_hbm.at[ic], bc, sc_).wait()

                def sub(j, c):
                    for f in range(w // 16):
                        sl = pl.ds(f * 16, 16)
                        d = br[j, sl] - bc[j, sl]
                        if (f + 1) * 16 <= din:
                            d = jnp.abs(d)
                        br[j, sl] = d
                    for f in cs_copy:
                        sl = pl.ds(f * 16, 16)
                        br[j, sl] = bc[j, sl]
                    return c

                lax.fori_loop(0, KSC, sub, 0)
                pltpu.sync_copy(br, out_hbm.at[pl.ds(base, KSC)])

        issue(0, slots[0])

        def body(i2, carry):
            i = 2 * i2
            issue(i + 1, slots[1])
            finish(i, slots[0])
            issue(i + 2, slots[0])
            finish(i + 1, slots[1])
            return carry

        lax.fori_loop(0, (iters + 1) // 2, body, 0)

    return k


def _sc_edge(t, rs, cs, din, dout, cs_copy=()):
    return _make_sc_edge(din, dout, cs_copy)(t, rs, cs)


def _make_sc_rowgather(mp):
    nch = NPAD // KSC
    iters = -(-nch // NWORK)
    mesh = plsc.VectorSubcoreMesh(core_axis_name="c", subcore_axis_name="s")

    @functools.partial(
        pl.kernel,
        out_type=jax.ShapeDtypeStruct((NPAD, mp), jnp.float32),
        mesh=mesh,
        scratch_types=[
            pltpu.VMEM((KSC,), jnp.int32),
            pltpu.VMEM((KSC, mp), jnp.float32),
            pltpu.SemaphoreType.DMA,
        ],
    )
    def k(src_hbm, idx_hbm, out_hbm, iv, buf, sem):
        wid = lax.axis_index("s") * 2 + lax.axis_index("c")

        def body(i, carry):
            cidx = wid + NWORK * i

            @pl.when(cidx < nch)
            def _():
                base = cidx * KSC
                pltpu.sync_copy(idx_hbm.at[pl.ds(base, KSC)], iv)
                pltpu.async_copy(src_hbm.at[iv], buf, sem).wait()
                pltpu.sync_copy(buf, out_hbm.at[pl.ds(base, KSC)])

            return carry

        lax.fori_loop(0, iters, body, 0)

    return k


def _sc_rowgather(src, idx):
    return _make_sc_rowgather(src.shape[1])(src, idx)


def _prep1_body(din, dout, wp, hn_ref, wt_ref, wb_ref, ew_ref, st_ref,
                t_ref, a_ref):
    hn = hn_ref[...]
    t_ref[:, :din] = hn
    t_ref[:, din:din + dout] = jnp.dot(hn, wb_ref[...],
                                       preferred_element_type=jnp.float32)
    nb = hn.shape[0]
    t_ref[:, din + dout:din + dout + 16] = jnp.broadcast_to(
        ew_ref[...], (nb, 16))
    t_ref[:, din + dout + 16:din + dout + 32] = jnp.broadcast_to(
        st_ref[...], (nb, 16))
    if wp > din + dout + 32:
        t_ref[:, din + dout + 32:] = jnp.zeros(
            (nb, wp - din - dout - 32), jnp.float32)
    a_ref[...] = jnp.dot(_lr(C_BN * hn), wt_ref[...],
                         preferred_element_type=jnp.float32)


def _prep1(hn, wt, wb, ew1, st1):
    din, dout = wt.shape
    wp = -(-(din + dout + 32) // 128) * 128
    grid = N // NBLK
    return pl.pallas_call(
        functools.partial(_prep1_body, din, dout, wp),
        grid=(grid,),
        in_specs=[
            pl.BlockSpec((NBLK, din), lambda i: (i, 0)),
            pl.BlockSpec((din, dout), lambda i: (0, 0)),
            pl.BlockSpec((din, dout), lambda i: (0, 0)),
            pl.BlockSpec((NBLK, 1), lambda i: (i, 0)),
            pl.BlockSpec((NBLK, 1), lambda i: (i, 0)),
        ],
        out_specs=[
            pl.BlockSpec((NBLK, wp), lambda i: (i, 0)),
            pl.BlockSpec((NBLK, dout), lambda i: (i, 0)),
        ],
        out_shape=[
            jax.ShapeDtypeStruct((N, wp), jnp.float32),
            jax.ShapeDtypeStruct((N, dout), jnp.float32),
        ],
    )(hn, wt, wb, ew1, st1)


def _prep_agg_body(din, dout, wp, ap_ref, m_ref, cnt_ref, wt_ref, wb_ref,
                   t_ref, a_ref, agg_ref):
    hn = jnp.where(cnt_ref[...] > 0, ap_ref[...] + m_ref[...], 0.0)
    agg_ref[...] = hn
    t_ref[:, :din] = hn
    t_ref[:, din:din + dout] = jnp.dot(hn, wb_ref[...],
                                       preferred_element_type=jnp.float32)
    if wp > din + dout:
        t_ref[:, din + dout:] = jnp.zeros(
            (hn.shape[0], wp - din - dout), jnp.float32)
    a_ref[...] = jnp.dot(_lr(C_BN * hn), wt_ref[...],
                         preferred_element_type=jnp.float32)


def _prep_agg(aprev, mpad, cntf, wt, wb):
    din, dout = wt.shape
    wp = -(-(din + dout) // 128) * 128
    grid = N // NBLK
    return pl.pallas_call(
        functools.partial(_prep_agg_body, din, dout, wp),
        grid=(grid,),
        in_specs=[
            pl.BlockSpec((NBLK, din), lambda i: (i, 0)),
            pl.BlockSpec((NBLK, din), lambda i: (i, 0)),
            pl.BlockSpec((NBLK, 1), lambda i: (i, 0)),
            pl.BlockSpec((din, dout), lambda i: (0, 0)),
            pl.BlockSpec((din, dout), lambda i: (0, 0)),
        ],
        out_specs=[
            pl.BlockSpec((NBLK, wp), lambda i: (i, 0)),
            pl.BlockSpec((NBLK, dout), lambda i: (i, 0)),
            pl.BlockSpec((NBLK, din), lambda i: (i, 0)),
        ],
        out_shape=[
            jax.ShapeDtypeStruct((N, wp), jnp.float32),
            jax.ShapeDtypeStruct((N, dout), jnp.float32),
            jax.ShapeDtypeStruct((N, din), jnp.float32),
        ],
    )(aprev, mpad, cntf, wt, wb)


def _edge_body(din, dout, mp, dxdu_ref, aux_ref, wb_ref, out_ref, carry_h):
    i = pl.program_id(0)

    @pl.when(i == 0)
    def _():
        carry_h[...] = jnp.zeros_like(carry_h)

    blk = dxdu_ref[...]
    dx = jnp.abs(blk[:, :din])
    du = blk[:, din:din + dout]
    ew = aux_ref[:, 0:1]
    startf = aux_ref[:, 1:2]
    m = ew * (C6 * du + C4 * jnp.dot(
        dx, wb_ref[...], preferred_element_type=jnp.float32))

    rowi = lax.broadcasted_iota(jnp.int32, (EBLK, 1), 0).astype(jnp.float32)
    gbase = (i * EBLK).astype(jnp.float32)
    posv = rowi + gbase - startf
    pr = jnp.minimum(posv, rowi)

    s = 1
    while s < EBLK:
        hp = pltpu.roll(m, s, 0)
        m = jnp.where(pr >= s, jnp.maximum(m, hp), m)
        s *= 2

    m = jnp.where(posv > rowi, jnp.maximum(m, carry_h[...]), m)

    carry_h[...] = m[EBLK - 1:EBLK, :]
    out_ref[:, :dout] = m
    if mp > dout:
        out_ref[:, dout:] = jnp.zeros((EBLK, mp - dout), jnp.float32)


def _edge_scan(dxdu, aux, wb):
    din, dout = wb.shape
    wp = dxdu.shape[1]
    mp = max(dout, 128)
    grid = E // EBLK
    return pl.pallas_call(
        functools.partial(_edge_body, din, dout, mp),
        grid=(grid,),
        in_specs=[
            pl.BlockSpec((EBLK, wp), lambda i: (i, 0)),
            pl.BlockSpec((EBLK, 2), lambda i: (i, 0)),
            pl.BlockSpec((din, dout), lambda i: (0, 0)),
        ],
        out_specs=pl.BlockSpec((EBLK, mp), lambda i: (i, 0)),
        out_shape=jax.ShapeDtypeStruct((E, mp), jnp.float32),
        scratch_shapes=[
            pltpu.VMEM((1, dout), jnp.float32),
        ],
    )(dxdu, aux, wb)


def _joint_body(a1, a2, a3, a4p, m4, cnt, w1, w2, w3, w4, wl1, wl2, wl3,
                cl1, cl2, cl3, out_ref, mx_s, sm_s):
    i = pl.program_id(0)
    nb = pl.num_programs(0)
    agg4 = jnp.where(cnt[...] > 0, a4p[...] + m4[...], 0.0)
    hj = (jnp.dot(_lr(C_BN * a1[...]), w1[...],
                  preferred_element_type=jnp.float32)
          + jnp.dot(_lr(C_BN * a2[...]), w2[...],
                    preferred_element_type=jnp.float32)
          + jnp.dot(_lr(C_BN * a3[...]), w3[...],
                    preferred_element_type=jnp.float32)
          + jnp.dot(_lr(C_BN * agg4), w4[...],
                    preferred_element_type=jnp.float32))
    bm = jnp.max(hj, axis=0, keepdims=True)
    bs = jnp.sum(hj, axis=0, keepdims=True)

    @pl.when(i == 0)
    def _():
        mx_s[...] = bm
        sm_s[...] = bs

    @pl.when(i > 0)
    def _():
        mx_s[...] = jnp.maximum(mx_s[...], bm)
        sm_s[...] = sm_s[...] + bs

    @pl.when(i == nb - 1)
    def _():
        g = jnp.concatenate([mx_s[...], sm_s[...] * (1.0 / N)], axis=1)
        g = jnp.dot(_lr(C_BN * g), wl1[...],
                    preferred_element_type=jnp.float32) + cl1[...]
        g = jnp.dot(_lr(C_BN * g), wl2[...],
                    preferred_element_type=jnp.float32) + cl2[...]
        g = jnp.dot(_lr(C_BN * g), wl3[...],
                    preferred_element_type=jnp.float32) + cl3[...]
        out_ref[...] = g


def _joint(aggs, a4, m4pad, cntf, wjs, p):
    grid = N // NBLK
    douts = [a.shape[1] for a in aggs] + [a4.shape[1]]
    in_specs = [pl.BlockSpec((NBLK, d), lambda i: (i, 0)) for d in douts]
    in_specs += [
        pl.BlockSpec((NBLK, a4.shape[1]), lambda i: (i, 0)),
        pl.BlockSpec((NBLK, 1), lambda i: (i, 0)),
    ]
    in_specs += [pl.BlockSpec((d, 1024), lambda i: (0, 0)) for d in douts]
    in_specs += [
        pl.BlockSpec((2048, 512), lambda i: (0, 0)),
        pl.BlockSpec((512, 256), lambda i: (0, 0)),
        pl.BlockSpec((256, NC), lambda i: (0, 0)),
        pl.BlockSpec((1, 512), lambda i: (0, 0)),
        pl.BlockSpec((1, 256), lambda i: (0, 0)),
        pl.BlockSpec((1, NC), lambda i: (0, 0)),
    ]
    return pl.pallas_call(
        _joint_body,
        grid=(grid,),
        in_specs=in_specs,
        out_specs=pl.BlockSpec((1, NC), lambda i: (0, 0)),
        out_shape=jax.ShapeDtypeStruct((1, NC), jnp.float32),
        scratch_shapes=[
            pltpu.VMEM((1, 1024), jnp.float32),
            pltpu.VMEM((1, 1024), jnp.float32),
        ],
    )(*aggs, a4, m4pad, cntf, *wjs,
      p['Wl1'], p['Wl2'], p['Wl3'],
      p['cl1'].reshape(1, -1), p['cl2'].reshape(1, -1),
      p['cl3'].reshape(1, -1))


def kernel(x, params, edge_index):
    p = params
    row = edge_index[0].astype(jnp.int32)
    col = edge_index[1].astype(jnp.int32)

    deg = jax.ops.segment_sum(jnp.ones((E,), jnp.float32), row,
                              num_segments=N)
    ewn = (1.0 / jnp.maximum(deg, 1.0))[:, None]

    cs, rs = lax.sort((col, row), num_keys=1)
    cnt = jax.ops.segment_sum(jnp.ones((E,), jnp.int32), cs,
                              num_segments=N, indices_are_sorted=True)
    end = jnp.cumsum(cnt)
    startf = (end - cnt).astype(jnp.float32)[:, None]
    cntf = cnt.astype(jnp.float32)[:, None]
    idxend = jnp.concatenate(
        [jnp.clip(end - 1, 0, E - 1),
         jnp.zeros((NPAD - N,), jnp.int32)])

    dins = (256, 64, 64, 128)
    wkeys = ('W1', 'W2', 'W3', 'W4')

    w = p['W1']
    wt, wb = w[:256], w[256:]
    t, a = _prep1(x, wt, wb, ewn, startf)
    dxdu = _sc_edge(t, rs, cs, 256, 64, cs_copy=(21,))
    aux = jnp.concatenate([dxdu[:, 320:321], dxdu[:, 336:337]], axis=1)
    mscan = _edge_scan(dxdu, aux, wb)
    mpad = _sc_rowgather(mscan, idxend)
    aggs = []
    aprev = a

    for li in (1, 2, 3):
        w = p[wkeys[li]]
        din = dins[li]
        wt, wb = w[:din], w[din:]
        t, a, agg = _prep_agg(aprev, mpad[:N, :din], cntf, wt, wb)
        aggs.append(agg)
        dxdu = _sc_edge(t, rs, cs, din, wb.shape[1])
        mscan = _edge_scan(dxdu, aux, wb)
        mpad = _sc_rowgather(mscan, idxend)
        aprev = a

    wj = p['Wj']
    wjs = [wj[0:64], wj[64:128], wj[128:256], wj[256:512]]
    return _joint(aggs, aprev, mpad, cntf, wjs, p)


# speedup vs baseline: 1.0803x; 1.0626x over previous
import functools

import jax
import jax.numpy as jnp
import numpy as np
from jax import lax
from jax.experimental import pallas as pl
from jax.experimental.pallas import tpu as pltpu
from jax.experimental.pallas import tpu_sc as plsc

N = 10000
E = 160000
NC = 40

C_BN = float(1.0 / np.sqrt(1.0 + 1e-5))
C6 = 0.6 * C_BN
C4 = 0.4 * C_BN

EBLK = 1280
NBLK = 1000
KSC = 64
NWORK = 32
NPAD = 10048


def _lr(v):
    return jnp.where(v >= 0, v, 0.2 * v)


def _make_sc_edge(din, dout, cs_copy):
    w = din + dout
    wp = -(-w // 128) * 128
    nch = E // KSC
    iters = -(-nch // NWORK)
    mesh = plsc.VectorSubcoreMesh(core_axis_name="c", subcore_axis_name="s")

    @functools.partial(
        pl.kernel,
        out_type=jax.ShapeDtypeStruct((E, wp), jnp.float32),
        mesh=mesh,
        scratch_types=[
            pltpu.VMEM((KSC,), jnp.int32),
            pltpu.VMEM((KSC,), jnp.int32),
            pltpu.VMEM((KSC,), jnp.int32),
            pltpu.VMEM((KSC,), jnp.int32),
            pltpu.VMEM((KSC, wp), jnp.float32),
            pltpu.VMEM((KSC, wp), jnp.float32),
            pltpu.VMEM((KSC, wp), jnp.float32),
            pltpu.VMEM((KSC, wp), jnp.float32),
            pltpu.SemaphoreType.DMA,
            pltpu.SemaphoreType.DMA,
            pltpu.SemaphoreType.DMA,
            pltpu.SemaphoreType.DMA,
        ],
    )
    def k(t_hbm, rs_hbm, cs_hbm, out_hbm,
          ir0, ic0, ir1, ic1, br0, bc0, br1, bc1, sr0, sc0, sr1, sc1):
        wid = lax.axis_index("s") * 2 + lax.axis_index("c")
        slots = ((ir0, ic0, br0, bc0, sr0, sc0),
                 (ir1, ic1, br1, bc1, sr1, sc1))

        def issue(i, slot):
            ir, ic, br, bc, sr, sc_ = slot
            cidx = wid + NWORK * i

            @pl.when(cidx < nch)
            def _():
                base = cidx * KSC
                pltpu.sync_copy(rs_hbm.at[pl.ds(base, KSC)], ir)
                pltpu.sync_copy(cs_hbm.at[pl.ds(base, KSC)], ic)
                pltpu.async_copy(t_hbm.at[ir], br, sr)
                pltpu.async_copy(t_hbm.at[ic], bc, sc_)

        def finish(i, slot):
            ir, ic, br, bc, sr, sc_ = slot
            cidx = wid + NWORK * i

            @pl.when(cidx < nch)
            def _():
                base = cidx * KSC
                pltpu.make_async_copy(t_hbm.at[ir], br, sr).wait()
                pltpu.make_async_copy(t_hbm.at[ic], bc, sc_).wait()

                def sub(j, c):
                    for f in range(w // 16):
                        sl = pl.ds(f * 16, 16)
                        d = br[j, sl] - bc[j, sl]
                        if (f + 1) * 16 <= din:
                            d = jnp.abs(d)
                        br[j, sl] = d
                    for f in cs_copy:
                        sl = pl.ds(f * 16, 16)
                        br[j, sl] = bc[j, sl]
                    return c

                lax.fori_loop(0, KSC, sub, 0)
                pltpu.sync_copy(br, out_hbm.at[pl.ds(base, KSC)])

        issue(0, slots[0])

        def body(i2, carry):
            i = 2 * i2
            issue(i + 1, slots[1])
            finish(i, slots[0])
            issue(i + 2, slots[0])
            finish(i + 1, slots[1])
            return carry

        lax.fori_loop(0, (iters + 1) // 2, body, 0)

    return k


def _sc_edge(t, rs, cs, din, dout, cs_copy=()):
    return _make_sc_edge(din, dout, cs_copy)(t, rs, cs)


def _make_sc_rowgather(mp):
    nch = NPAD // KSC
    iters = -(-nch // NWORK)
    mesh = plsc.VectorSubcoreMesh(core_axis_name="c", subcore_axis_name="s")

    @functools.partial(
        pl.kernel,
        out_type=jax.ShapeDtypeStruct((NPAD, mp), jnp.float32),
        mesh=mesh,
        scratch_types=[
            pltpu.VMEM((KSC,), jnp.int32),
            pltpu.VMEM((KSC, mp), jnp.float32),
            pltpu.SemaphoreType.DMA,
        ],
    )
    def k(src_hbm, idx_hbm, out_hbm, iv, buf, sem):
        wid = lax.axis_index("s") * 2 + lax.axis_index("c")

        def body(i, carry):
            cidx = wid + NWORK * i

            @pl.when(cidx < nch)
            def _():
                base = cidx * KSC
                pltpu.sync_copy(idx_hbm.at[pl.ds(base, KSC)], iv)
                pltpu.async_copy(src_hbm.at[iv], buf, sem).wait()
                pltpu.sync_copy(buf, out_hbm.at[pl.ds(base, KSC)])

            return carry

        lax.fori_loop(0, iters, body, 0)

    return k


def _sc_rowgather(src, idx):
    return _make_sc_rowgather(src.shape[1])(src, idx)


def _prep1_body(din, dout, wp, hn_ref, wt_ref, wb_ref, ew_ref, st_ref,
                t_ref, a_ref):
    hn = hn_ref[...]
    t_ref[:, :din] = hn
    t_ref[:, din:din + dout] = jnp.dot(hn, wb_ref[...],
                                       preferred_element_type=jnp.float32)
    nb = hn.shape[0]
    t_ref[:, din + dout:din + dout + 16] = jnp.broadcast_to(
        ew_ref[...], (nb, 16))
    t_ref[:, din + dout + 16:din + dout + 32] = jnp.broadcast_to(
        st_ref[...], (nb, 16))
    if wp > din + dout + 32:
        t_ref[:, din + dout + 32:] = jnp.zeros(
            (nb, wp - din - dout - 32), jnp.float32)
    a_ref[...] = jnp.dot(_lr(C_BN * hn), wt_ref[...],
                         preferred_element_type=jnp.float32)


def _prep1(hn, wt, wb, ew1, st1):
    din, dout = wt.shape
    wp = -(-(din + dout + 32) // 128) * 128
    grid = N // NBLK
    return pl.pallas_call(
        functools.partial(_prep1_body, din, dout, wp),
        grid=(grid,),
        in_specs=[
            pl.BlockSpec((NBLK, din), lambda i: (i, 0)),
            pl.BlockSpec((din, dout), lambda i: (0, 0)),
            pl.BlockSpec((din, dout), lambda i: (0, 0)),
            pl.BlockSpec((NBLK, 1), lambda i: (i, 0)),
            pl.BlockSpec((NBLK, 1), lambda i: (i, 0)),
        ],
        out_specs=[
            pl.BlockSpec((NBLK, wp), lambda i: (i, 0)),
            pl.BlockSpec((NBLK, dout), lambda i: (i, 0)),
        ],
        out_shape=[
            jax.ShapeDtypeStruct((N, wp), jnp.float32),
            jax.ShapeDtypeStruct((N, dout), jnp.float32),
        ],
    )(hn, wt, wb, ew1, st1)


def _prep_agg_body(din, dout, wp, ap_ref, m_ref, cnt_ref, wt_ref, wb_ref,
                   t_ref, a_ref, agg_ref):
    hn = jnp.where(cnt_ref[...] > 0, ap_ref[...] + m_ref[...], 0.0)
    agg_ref[...] = hn
    t_ref[:, :din] = hn
    t_ref[:, din:din + dout] = jnp.dot(hn, wb_ref[...],
                                       preferred_element_type=jnp.float32)
    if wp > din + dout:
        t_ref[:, din + dout:] = jnp.zeros(
            (hn.shape[0], wp - din - dout), jnp.float32)
    a_ref[...] = jnp.dot(_lr(C_BN * hn), wt_ref[...],
                         preferred_element_type=jnp.float32)


def _prep_agg(aprev, mpad, cntf, wt, wb):
    din, dout = wt.shape
    wp = -(-(din + dout) // 128) * 128
    grid = N // NBLK
    return pl.pallas_call(
        functools.partial(_prep_agg_body, din, dout, wp),
        grid=(grid,),
        in_specs=[
            pl.BlockSpec((NBLK, din), lambda i: (i, 0)),
            pl.BlockSpec((NBLK, din), lambda i: (i, 0)),
            pl.BlockSpec((NBLK, 1), lambda i: (i, 0)),
            pl.BlockSpec((din, dout), lambda i: (0, 0)),
            pl.BlockSpec((din, dout), lambda i: (0, 0)),
        ],
        out_specs=[
            pl.BlockSpec((NBLK, wp), lambda i: (i, 0)),
            pl.BlockSpec((NBLK, dout), lambda i: (i, 0)),
            pl.BlockSpec((NBLK, din), lambda i: (i, 0)),
        ],
        out_shape=[
            jax.ShapeDtypeStruct((N, wp), jnp.float32),
            jax.ShapeDtypeStruct((N, dout), jnp.float32),
            jax.ShapeDtypeStruct((N, din), jnp.float32),
        ],
    )(aprev, mpad, cntf, wt, wb)


def _edge_body(din, dout, mp, dxdu_ref, aux_ref, wb_ref, out_ref, carry_h):
    i = pl.program_id(0)

    @pl.when(i == 0)
    def _():
        carry_h[...] = jnp.zeros_like(carry_h)

    blk = dxdu_ref[...]
    dx = jnp.abs(blk[:, :din])
    du = blk[:, din:din + dout]
    ew = aux_ref[:, 0:1]
    startf = aux_ref[:, 1:2]
    m = ew * (C6 * du + C4 * jnp.dot(
        dx, wb_ref[...], preferred_element_type=jnp.float32))

    rowi = lax.broadcasted_iota(jnp.int32, (EBLK, 1), 0).astype(jnp.float32)
    gbase = (i * EBLK).astype(jnp.float32)
    posv = rowi + gbase - startf
    pr = jnp.minimum(posv, rowi)

    s = 1
    while s < EBLK:
        hp = jnp.concatenate(
            [jnp.zeros((s, dout), jnp.float32), m[:-s]], axis=0)
        m = jnp.where(pr >= s, jnp.maximum(m, hp), m)
        s *= 2

    m = jnp.where(posv > rowi, jnp.maximum(m, carry_h[...]), m)

    carry_h[...] = m[EBLK - 1:EBLK, :]
    out_ref[:, :dout] = m
    if mp > dout:
        out_ref[:, dout:] = jnp.zeros((EBLK, mp - dout), jnp.float32)


def _edge_scan(dxdu, aux, wb):
    din, dout = wb.shape
    wp = dxdu.shape[1]
    mp = max(dout, 128)
    grid = E // EBLK
    return pl.pallas_call(
        functools.partial(_edge_body, din, dout, mp),
        grid=(grid,),
        in_specs=[
            pl.BlockSpec((EBLK, wp), lambda i: (i, 0)),
            pl.BlockSpec((EBLK, 2), lambda i: (i, 0)),
            pl.BlockSpec((din, dout), lambda i: (0, 0)),
        ],
        out_specs=pl.BlockSpec((EBLK, mp), lambda i: (i, 0)),
        out_shape=jax.ShapeDtypeStruct((E, mp), jnp.float32),
        scratch_shapes=[
            pltpu.VMEM((1, dout), jnp.float32),
        ],
    )(dxdu, aux, wb)


def _joint_body(a1, a2, a3, a4p, m4, cnt, w1, w2, w3, w4, wl1, wl2, wl3,
                cl1, cl2, cl3, out_ref, mx_s, sm_s):
    i = pl.program_id(0)
    nb = pl.num_programs(0)
    agg4 = jnp.where(cnt[...] > 0, a4p[...] + m4[...], 0.0)
    hj = (jnp.dot(_lr(C_BN * a1[...]), w1[...],
                  preferred_element_type=jnp.float32)
          + jnp.dot(_lr(C_BN * a2[...]), w2[...],
                    preferred_element_type=jnp.float32)
          + jnp.dot(_lr(C_BN * a3[...]), w3[...],
                    preferred_element_type=jnp.float32)
          + jnp.dot(_lr(C_BN * agg4), w4[...],
                    preferred_element_type=jnp.float32))
    bm = jnp.max(hj, axis=0, keepdims=True)
    bs = jnp.sum(hj, axis=0, keepdims=True)

    @pl.when(i == 0)
    def _():
        mx_s[...] = bm
        sm_s[...] = bs

    @pl.when(i > 0)
    def _():
        mx_s[...] = jnp.maximum(mx_s[...], bm)
        sm_s[...] = sm_s[...] + bs

    @pl.when(i == nb - 1)
    def _():
        g = jnp.concatenate([mx_s[...], sm_s[...] * (1.0 / N)], axis=1)
        g = jnp.dot(_lr(C_BN * g), wl1[...],
                    preferred_element_type=jnp.float32) + cl1[...]
        g = jnp.dot(_lr(C_BN * g), wl2[...],
                    preferred_element_type=jnp.float32) + cl2[...]
        g = jnp.dot(_lr(C_BN * g), wl3[...],
                    preferred_element_type=jnp.float32) + cl3[...]
        out_ref[...] = g


def _joint(aggs, a4, m4pad, cntf, wjs, p):
    grid = N // NBLK
    douts = [a.shape[1] for a in aggs] + [a4.shape[1]]
    in_specs = [pl.BlockSpec((NBLK, d), lambda i: (i, 0)) for d in douts]
    in_specs += [
        pl.BlockSpec((NBLK, a4.shape[1]), lambda i: (i, 0)),
        pl.BlockSpec((NBLK, 1), lambda i: (i, 0)),
    ]
    in_specs += [pl.BlockSpec((d, 1024), lambda i: (0, 0)) for d in douts]
    in_specs += [
        pl.BlockSpec((2048, 512), lambda i: (0, 0)),
        pl.BlockSpec((512, 256), lambda i: (0, 0)),
        pl.BlockSpec((256, NC), lambda i: (0, 0)),
        pl.BlockSpec((1, 512), lambda i: (0, 0)),
        pl.BlockSpec((1, 256), lambda i: (0, 0)),
        pl.BlockSpec((1, NC), lambda i: (0, 0)),
    ]
    return pl.pallas_call(
        _joint_body,
        grid=(grid,),
        in_specs=in_specs,
        out_specs=pl.BlockSpec((1, NC), lambda i: (0, 0)),
        out_shape=jax.ShapeDtypeStruct((1, NC), jnp.float32),
        scratch_shapes=[
            pltpu.VMEM((1, 1024), jnp.float32),
            pltpu.VMEM((1, 1024), jnp.float32),
        ],
    )(*aggs, a4, m4pad, cntf, *wjs,
      p['Wl1'], p['Wl2'], p['Wl3'],
      p['cl1'].reshape(1, -1), p['cl2'].reshape(1, -1),
      p['cl3'].reshape(1, -1))


def kernel(x, params, edge_index):
    p = params
    row = edge_index[0].astype(jnp.int32)
    col = edge_index[1].astype(jnp.int32)

    deg = jax.ops.segment_sum(jnp.ones((E,), jnp.float32), row,
                              num_segments=N)
    ewn = (1.0 / jnp.maximum(deg, 1.0))[:, None]

    cs, rs = lax.sort((col, row), num_keys=1)
    cnt = jax.ops.segment_sum(jnp.ones((E,), jnp.int32), cs,
                              num_segments=N, indices_are_sorted=True)
    end = jnp.cumsum(cnt)
    startf = (end - cnt).astype(jnp.float32)[:, None]
    cntf = cnt.astype(jnp.float32)[:, None]
    idxend = jnp.concatenate(
        [jnp.clip(end - 1, 0, E - 1),
         jnp.zeros((NPAD - N,), jnp.int32)])

    dins = (256, 64, 64, 128)
    wkeys = ('W1', 'W2', 'W3', 'W4')

    w = p['W1']
    wt, wb = w[:256], w[256:]
    t, a = _prep1(x, wt, wb, ewn, startf)
    dxdu = _sc_edge(t, rs, cs, 256, 64, cs_copy=(21,))
    aux = jnp.concatenate([dxdu[:, 320:321], dxdu[:, 336:337]], axis=1)
    mscan = _edge_scan(dxdu, aux, wb)
    mpad = _sc_rowgather(mscan, idxend)
    aggs = []
    aprev = a

    for li in (1, 2, 3):
        w = p[wkeys[li]]
        din = dins[li]
        wt, wb = w[:din], w[din:]
        t, a, agg = _prep_agg(aprev, mpad[:N, :din], cntf, wt, wb)
        aggs.append(agg)
        dxdu = _sc_edge(t, rs, cs, din, wb.shape[1])
        mscan = _edge_scan(dxdu, aux, wb)
        mpad = _sc_rowgather(mscan, idxend)
        aprev = a

    wj = p['Wj']
    wjs = [wj[0:64], wj[64:128], wj[128:256], wj[256:512]]
    return _joint(aggs, aprev, mpad, cntf, wjs, p)


# speedup vs baseline: 1.0891x; 1.0081x over previous
import functools

import jax
import jax.numpy as jnp
import numpy as np
from jax import lax
from jax.experimental import pallas as pl
from jax.experimental.pallas import tpu as pltpu
from jax.experimental.pallas import tpu_sc as plsc

N = 10000
E = 160000
NC = 40

C_BN = float(1.0 / np.sqrt(1.0 + 1e-5))
C6 = 0.6 * C_BN
C4 = 0.4 * C_BN

EBLK = 1280
NBLK = 1000
KSC = 64
NWORK = 32
NPAD = 10048


def _lr(v):
    return jnp.where(v >= 0, v, 0.2 * v)


def _make_sc_edge(din, dout, cs_copy):
    w = din + dout
    wp = -(-w // 128) * 128
    ksc = 128 if wp <= 128 else KSC
    nch = E // ksc
    iters = -(-nch // NWORK)
    mesh = plsc.VectorSubcoreMesh(core_axis_name="c", subcore_axis_name="s")

    @functools.partial(
        pl.kernel,
        out_type=jax.ShapeDtypeStruct((E, wp), jnp.float32),
        mesh=mesh,
        scratch_types=[
            pltpu.VMEM((ksc,), jnp.int32),
            pltpu.VMEM((ksc,), jnp.int32),
            pltpu.VMEM((ksc,), jnp.int32),
            pltpu.VMEM((ksc,), jnp.int32),
            pltpu.VMEM((ksc, wp), jnp.float32),
            pltpu.VMEM((ksc, wp), jnp.float32),
            pltpu.VMEM((ksc, wp), jnp.float32),
            pltpu.VMEM((ksc, wp), jnp.float32),
            pltpu.SemaphoreType.DMA,
            pltpu.SemaphoreType.DMA,
            pltpu.SemaphoreType.DMA,
            pltpu.SemaphoreType.DMA,
        ],
    )
    def k(t_hbm, rs_hbm, cs_hbm, out_hbm,
          ir0, ic0, ir1, ic1, br0, bc0, br1, bc1, sr0, sc0, sr1, sc1):
        wid = lax.axis_index("s") * 2 + lax.axis_index("c")
        slots = ((ir0, ic0, br0, bc0, sr0, sc0),
                 (ir1, ic1, br1, bc1, sr1, sc1))

        def issue(i, slot):
            ir, ic, br, bc, sr, sc_ = slot
            cidx = wid + NWORK * i

            @pl.when(cidx < nch)
            def _():
                base = cidx * ksc
                pltpu.sync_copy(rs_hbm.at[pl.ds(base, ksc)], ir)
                pltpu.sync_copy(cs_hbm.at[pl.ds(base, ksc)], ic)
                pltpu.async_copy(t_hbm.at[ir], br, sr)
                pltpu.async_copy(t_hbm.at[ic], bc, sc_)

        def finish(i, slot):
            ir, ic, br, bc, sr, sc_ = slot
            cidx = wid + NWORK * i

            @pl.when(cidx < nch)
            def _():
                base = cidx * ksc
                pltpu.make_async_copy(t_hbm.at[ir], br, sr).wait()
                pltpu.make_async_copy(t_hbm.at[ic], bc, sc_).wait()

                def sub(j, c):
                    for f in range(w // 16):
                        sl = pl.ds(f * 16, 16)
                        d = br[j, sl] - bc[j, sl]
                        if (f + 1) * 16 <= din:
                            d = jnp.abs(d)
                        br[j, sl] = d
                    for f in cs_copy:
                        sl = pl.ds(f * 16, 16)
                        br[j, sl] = bc[j, sl]
                    return c

                lax.fori_loop(0, ksc, sub, 0)
                pltpu.sync_copy(br, out_hbm.at[pl.ds(base, ksc)])

        issue(0, slots[0])

        def body(i2, carry):
            i = 2 * i2
            issue(i + 1, slots[1])
            finish(i, slots[0])
            issue(i + 2, slots[0])
            finish(i + 1, slots[1])
            return carry

        lax.fori_loop(0, (iters + 1) // 2, body, 0)

    return k


def _sc_edge(t, rs, cs, din, dout, cs_copy=()):
    return _make_sc_edge(din, dout, cs_copy)(t, rs, cs)


def _make_sc_rowgather(mp):
    nch = NPAD // KSC
    iters = -(-nch // NWORK)
    mesh = plsc.VectorSubcoreMesh(core_axis_name="c", subcore_axis_name="s")

    @functools.partial(
        pl.kernel,
        out_type=jax.ShapeDtypeStruct((NPAD, mp), jnp.float32),
        mesh=mesh,
        scratch_types=[
            pltpu.VMEM((KSC,), jnp.int32),
            pltpu.VMEM((KSC, mp), jnp.float32),
            pltpu.SemaphoreType.DMA,
        ],
    )
    def k(src_hbm, idx_hbm, out_hbm, iv, buf, sem):
        wid = lax.axis_index("s") * 2 + lax.axis_index("c")

        def body(i, carry):
            cidx = wid + NWORK * i

            @pl.when(cidx < nch)
            def _():
                base = cidx * KSC
                pltpu.sync_copy(idx_hbm.at[pl.ds(base, KSC)], iv)
                pltpu.async_copy(src_hbm.at[iv], buf, sem).wait()
                pltpu.sync_copy(buf, out_hbm.at[pl.ds(base, KSC)])

            return carry

        lax.fori_loop(0, iters, body, 0)

    return k


def _sc_rowgather(src, idx):
    return _make_sc_rowgather(src.shape[1])(src, idx)


def _prep1_body(din, dout, wp, hn_ref, wt_ref, wb_ref, ew_ref, st_ref,
                t_ref, a_ref):
    hn = hn_ref[...]
    t_ref[:, :din] = hn
    t_ref[:, din:din + dout] = jnp.dot(hn, wb_ref[...],
                                       preferred_element_type=jnp.float32)
    nb = hn.shape[0]
    t_ref[:, din + dout:din + dout + 16] = jnp.broadcast_to(
        ew_ref[...], (nb, 16))
    t_ref[:, din + dout + 16:din + dout + 32] = jnp.broadcast_to(
        st_ref[...], (nb, 16))
    if wp > din + dout + 32:
        t_ref[:, din + dout + 32:] = jnp.zeros(
            (nb, wp - din - dout - 32), jnp.float32)
    a_ref[...] = jnp.dot(_lr(C_BN * hn), wt_ref[...],
                         preferred_element_type=jnp.float32)


def _prep1(hn, wt, wb, ew1, st1):
    din, dout = wt.shape
    wp = -(-(din + dout + 32) // 128) * 128
    grid = N // NBLK
    return pl.pallas_call(
        functools.partial(_prep1_body, din, dout, wp),
        grid=(grid,),
        in_specs=[
            pl.BlockSpec((NBLK, din), lambda i: (i, 0)),
            pl.BlockSpec((din, dout), lambda i: (0, 0)),
            pl.BlockSpec((din, dout), lambda i: (0, 0)),
            pl.BlockSpec((NBLK, 1), lambda i: (i, 0)),
            pl.BlockSpec((NBLK, 1), lambda i: (i, 0)),
        ],
        out_specs=[
            pl.BlockSpec((NBLK, wp), lambda i: (i, 0)),
            pl.BlockSpec((NBLK, dout), lambda i: (i, 0)),
        ],
        out_shape=[
            jax.ShapeDtypeStruct((N, wp), jnp.float32),
            jax.ShapeDtypeStruct((N, dout), jnp.float32),
        ],
    )(hn, wt, wb, ew1, st1)


def _prep_agg_body(din, dout, wp, ap_ref, m_ref, cnt_ref, wt_ref, wb_ref,
                   t_ref, a_ref, agg_ref):
    hn = jnp.where(cnt_ref[...] > 0, ap_ref[...] + m_ref[...], 0.0)
    agg_ref[...] = hn
    t_ref[:, :din] = hn
    t_ref[:, din:din + dout] = jnp.dot(hn, wb_ref[...],
                                       preferred_element_type=jnp.float32)
    if wp > din + dout:
        t_ref[:, din + dout:] = jnp.zeros(
            (hn.shape[0], wp - din - dout), jnp.float32)
    a_ref[...] = jnp.dot(_lr(C_BN * hn), wt_ref[...],
                         preferred_element_type=jnp.float32)


def _prep_agg(aprev, mpad, cntf, wt, wb):
    din, dout = wt.shape
    wp = -(-(din + dout) // 128) * 128
    grid = N // NBLK
    return pl.pallas_call(
        functools.partial(_prep_agg_body, din, dout, wp),
        grid=(grid,),
        in_specs=[
            pl.BlockSpec((NBLK, din), lambda i: (i, 0)),
            pl.BlockSpec((NBLK, din), lambda i: (i, 0)),
            pl.BlockSpec((NBLK, 1), lambda i: (i, 0)),
            pl.BlockSpec((din, dout), lambda i: (0, 0)),
            pl.BlockSpec((din, dout), lambda i: (0, 0)),
        ],
        out_specs=[
            pl.BlockSpec((NBLK, wp), lambda i: (i, 0)),
            pl.BlockSpec((NBLK, dout), lambda i: (i, 0)),
            pl.BlockSpec((NBLK, din), lambda i: (i, 0)),
        ],
        out_shape=[
            jax.ShapeDtypeStruct((N, wp), jnp.float32),
            jax.ShapeDtypeStruct((N, dout), jnp.float32),
            jax.ShapeDtypeStruct((N, din), jnp.float32),
        ],
    )(aprev, mpad, cntf, wt, wb)


def _edge_body(din, dout, mp, dxdu_ref, aux_ref, wb_ref, out_ref, carry_h):
    i = pl.program_id(0)

    @pl.when(i == 0)
    def _():
        carry_h[...] = jnp.zeros_like(carry_h)

    blk = dxdu_ref[...]
    dx = jnp.abs(blk[:, :din])
    du = blk[:, din:din + dout]
    ew = aux_ref[:, 0:1]
    startf = aux_ref[:, 1:2]
    m = ew * (C6 * du + C4 * jnp.dot(
        dx, wb_ref[...], preferred_element_type=jnp.float32))

    rowi = lax.broadcasted_iota(jnp.int32, (EBLK, 1), 0).astype(jnp.float32)
    gbase = (i * EBLK).astype(jnp.float32)
    posv = rowi + gbase - startf
    pr = jnp.minimum(posv, rowi)

    s = 1
    while s < EBLK:
        hp = jnp.concatenate(
            [jnp.zeros((s, dout), jnp.float32), m[:-s]], axis=0)
        m = jnp.where(pr >= s, jnp.maximum(m, hp), m)
        s *= 2

    m = jnp.where(posv > rowi, jnp.maximum(m, carry_h[...]), m)

    carry_h[...] = m[EBLK - 1:EBLK, :]
    out_ref[:, :dout] = m
    if mp > dout:
        out_ref[:, dout:] = jnp.zeros((EBLK, mp - dout), jnp.float32)


def _edge_scan(dxdu, aux, wb):
    din, dout = wb.shape
    wp = dxdu.shape[1]
    mp = max(dout, 128)
    grid = E // EBLK
    return pl.pallas_call(
        functools.partial(_edge_body, din, dout, mp),
        grid=(grid,),
        in_specs=[
            pl.BlockSpec((EBLK, wp), lambda i: (i, 0)),
            pl.BlockSpec((EBLK, 2), lambda i: (i, 0)),
            pl.BlockSpec((din, dout), lambda i: (0, 0)),
        ],
        out_specs=pl.BlockSpec((EBLK, mp), lambda i: (i, 0)),
        out_shape=jax.ShapeDtypeStruct((E, mp), jnp.float32),
        scratch_shapes=[
            pltpu.VMEM((1, dout), jnp.float32),
        ],
    )(dxdu, aux, wb)


def _joint_body(a1, a2, a3, a4p, m4, cnt, w1, w2, w3, w4, wl1, wl2, wl3,
                cl1, cl2, cl3, out_ref, mx_s, sm_s):
    i = pl.program_id(0)
    nb = pl.num_programs(0)
    agg4 = jnp.where(cnt[...] > 0, a4p[...] + m4[...], 0.0)
    hj = (jnp.dot(_lr(C_BN * a1[...]), w1[...],
                  preferred_element_type=jnp.float32)
          + jnp.dot(_lr(C_BN * a2[...]), w2[...],
                    preferred_element_type=jnp.float32)
          + jnp.dot(_lr(C_BN * a3[...]), w3[...],
                    preferred_element_type=jnp.float32)
          + jnp.dot(_lr(C_BN * agg4), w4[...],
                    preferred_element_type=jnp.float32))
    bm = jnp.max(hj, axis=0, keepdims=True)
    bs = jnp.sum(hj, axis=0, keepdims=True)

    @pl.when(i == 0)
    def _():
        mx_s[...] = bm
        sm_s[...] = bs

    @pl.when(i > 0)
    def _():
        mx_s[...] = jnp.maximum(mx_s[...], bm)
        sm_s[...] = sm_s[...] + bs

    @pl.when(i == nb - 1)
    def _():
        g = jnp.concatenate([mx_s[...], sm_s[...] * (1.0 / N)], axis=1)
        g = jnp.dot(_lr(C_BN * g), wl1[...],
                    preferred_element_type=jnp.float32) + cl1[...]
        g = jnp.dot(_lr(C_BN * g), wl2[...],
                    preferred_element_type=jnp.float32) + cl2[...]
        g = jnp.dot(_lr(C_BN * g), wl3[...],
                    preferred_element_type=jnp.float32) + cl3[...]
        out_ref[...] = g


def _joint(aggs, a4, m4pad, cntf, wjs, p):
    grid = N // NBLK
    douts = [a.shape[1] for a in aggs] + [a4.shape[1]]
    in_specs = [pl.BlockSpec((NBLK, d), lambda i: (i, 0)) for d in douts]
    in_specs += [
        pl.BlockSpec((NBLK, a4.shape[1]), lambda i: (i, 0)),
        pl.BlockSpec((NBLK, 1), lambda i: (i, 0)),
    ]
    in_specs += [pl.BlockSpec((d, 1024), lambda i: (0, 0)) for d in douts]
    in_specs += [
        pl.BlockSpec((2048, 512), lambda i: (0, 0)),
        pl.BlockSpec((512, 256), lambda i: (0, 0)),
        pl.BlockSpec((256, NC), lambda i: (0, 0)),
        pl.BlockSpec((1, 512), lambda i: (0, 0)),
        pl.BlockSpec((1, 256), lambda i: (0, 0)),
        pl.BlockSpec((1, NC), lambda i: (0, 0)),
    ]
    return pl.pallas_call(
        _joint_body,
        grid=(grid,),
        in_specs=in_specs,
        out_specs=pl.BlockSpec((1, NC), lambda i: (0, 0)),
        out_shape=jax.ShapeDtypeStruct((1, NC), jnp.float32),
        scratch_shapes=[
            pltpu.VMEM((1, 1024), jnp.float32),
            pltpu.VMEM((1, 1024), jnp.float32),
        ],
    )(*aggs, a4, m4pad, cntf, *wjs,
      p['Wl1'], p['Wl2'], p['Wl3'],
      p['cl1'].reshape(1, -1), p['cl2'].reshape(1, -1),
      p['cl3'].reshape(1, -1))


def kernel(x, params, edge_index):
    p = params
    row = edge_index[0].astype(jnp.int32)
    col = edge_index[1].astype(jnp.int32)

    deg = jax.ops.segment_sum(jnp.ones((E,), jnp.float32), row,
                              num_segments=N)
    ewn = (1.0 / jnp.maximum(deg, 1.0))[:, None]

    cs, rs = lax.sort((col, row), num_keys=1)
    cnt = jax.ops.segment_sum(jnp.ones((E,), jnp.int32), cs,
                              num_segments=N, indices_are_sorted=True)
    end = jnp.cumsum(cnt)
    startf = (end - cnt).astype(jnp.float32)[:, None]
    cntf = cnt.astype(jnp.float32)[:, None]
    idxend = jnp.concatenate(
        [jnp.clip(end - 1, 0, E - 1),
         jnp.zeros((NPAD - N,), jnp.int32)])

    dins = (256, 64, 64, 128)
    wkeys = ('W1', 'W2', 'W3', 'W4')

    w = p['W1']
    wt, wb = w[:256], w[256:]
    t, a = _prep1(x, wt, wb, ewn, startf)
    dxdu = _sc_edge(t, rs, cs, 256, 64, cs_copy=(21,))
    aux = jnp.concatenate([dxdu[:, 320:321], dxdu[:, 336:337]], axis=1)
    mscan = _edge_scan(dxdu, aux, wb)
    mpad = _sc_rowgather(mscan, idxend)
    aggs = []
    aprev = a

    for li in (1, 2, 3):
        w = p[wkeys[li]]
        din = dins[li]
        wt, wb = w[:din], w[din:]
        t, a, agg = _prep_agg(aprev, mpad[:N, :din], cntf, wt, wb)
        aggs.append(agg)
        dxdu = _sc_edge(t, rs, cs, din, wb.shape[1])
        mscan = _edge_scan(dxdu, aux, wb)
        mpad = _sc_rowgather(mscan, idxend)
        aprev = a

    wj = p['Wj']
    wjs = [wj[0:64], wj[64:128], wj[128:256], wj[256:512]]
    return _joint(aggs, aprev, mpad, cntf, wjs, p)
